# baseline (speedup 1.0000x reference)
import jax
import jax.numpy as jnp
from jax.experimental import pallas as pl
from jax.experimental.pallas import tpu as pltpu

B = 8
Q = 900
T = 512
C = 400
K = 50
CQ = 128
NG = 8
QP = CQ * NG
BIG = 1 << 30


def _body(logits_ref, boxes_ref, ts_ref, wt_ref,
          scores_ref, labels_ref, boxesout_ref,
          p_ref, rm_ref, bx_ref, pm_ref):
    pid = pl.program_id(0)

    @pl.when(pid == 0)
    def _cast_pm():
        pm_ref[...] = wt_ref[...].astype(jnp.bfloat16)

    @pl.when(pid < NG)
    def _matmul_step():
        x = logits_ref[...].reshape(CQ * B, T)
        sig = jax.nn.sigmoid(x).astype(jnp.bfloat16)
        pm = pm_ref[...]
        p = jax.lax.dot_general(
            sig, pm, (((1,), (1,)), ((), ())),
            preferred_element_type=jnp.float32)
        p_ref[pl.ds(pid * CQ * B, CQ * B), :] = p
        rmc = jnp.max(p.reshape(CQ, B, C), axis=2)
        rm_ref[:, pl.ds(pid * CQ, CQ)] = jnp.transpose(rmc)

    @pl.when(pid == NG)
    def _extract_step():
        pb = boxes_ref[...]
        cxy = pb[:, 0:2, :]
        wh2 = pb[:, 2:4, :] * 0.5
        xyxy = jnp.concatenate([cxy - wh2, cxy + wh2], axis=1)
        for b in range(B):
            h = ts_ref[b, 0]
            w = ts_ref[b, 1]
            scale = jnp.concatenate(
                [jnp.full((1, Q), w, jnp.float32),
                 jnp.full((1, Q), h, jnp.float32)] * 2, axis=0)
            bx_ref[b] = xyxy[b] * scale

        qio = jax.lax.broadcasted_iota(jnp.int32, (B, QP), 1)
        cio = jax.lax.broadcasted_iota(jnp.int32, (B, C), 1)
        kio = jax.lax.broadcasted_iota(jnp.int32, (B, K), 1)
        rm = rm_ref[...]
        rm = jnp.where(qio >= Q, -1.0, rm)
        sc_acc = jnp.zeros((B, K), jnp.float32)
        lb_acc = jnp.zeros((B, K), jnp.int32)
        q_acc = jnp.zeros((B, K), jnp.int32)

        for k2 in range(K // 2):
            k = 2 * k2
            m1 = jnp.max(rm, axis=1, keepdims=True)
            q1v = jnp.min(jnp.where(rm == m1, qio, BIG),
                          axis=1, keepdims=True)
            rme = jnp.where(qio == q1v, -2.0, rm)
            m2 = jnp.max(rme, axis=1, keepdims=True)
            q2v = jnp.min(jnp.where(rme == m2, qio, BIG),
                          axis=1, keepdims=True)
            rowsA = []
            rowsB = []
            rsA = []
            rsB = []
            for b in range(B):
                ra = q1v[b, 0] * B + b
                rb = q2v[b, 0] * B + b
                rsA.append(ra)
                rsB.append(rb)
                rowsA.append(p_ref[pl.ds(ra, 1), :])
                rowsB.append(p_ref[pl.ds(rb, 1), :])
            rowsA = jnp.concatenate(rowsA, axis=0)
            rowsB = jnp.concatenate(rowsB, axis=0)
            c1 = jnp.min(jnp.where(rowsA == m1, cio, BIG),
                         axis=1, keepdims=True)
            nrowA = jnp.where(cio == c1, -1.0, rowsA)
            nm1 = jnp.max(nrowA, axis=1, keepdims=True)
            c1p = jnp.min(jnp.where(nrowA == nm1, cio, BIG),
                          axis=1, keepdims=True)
            c2 = jnp.min(jnp.where(rowsB == m2, cio, BIG),
                         axis=1, keepdims=True)
            flag = (nm1 > m2) | ((nm1 == m2) & (q1v < q2v))
            pick2v = jnp.where(flag, nm1, m2)
            pick2c = jnp.where(flag, c1p, c2)
            pick2q = jnp.where(flag, q1v, q2v)
            rowAf = jnp.where((cio == c1) | (flag & (cio == c1p)),
                              -1.0, rowsA)
            rowBf = jnp.where((~flag) & (cio == c2), -1.0, rowsB)
            for b in range(B):
                p_ref[pl.ds(rsA[b], 1), :] = rowAf[b:b + 1, :]
                p_ref[pl.ds(rsB[b], 1), :] = rowBf[b:b + 1, :]
            rmA = jnp.max(rowAf, axis=1, keepdims=True)
            rmB = jnp.max(rowBf, axis=1, keepdims=True)
            rm = jnp.where(qio == q1v, rmA, rm)
            rm = jnp.where(qio == q2v, rmB, rm)
            sc_acc = jnp.where(kio == k, m1, sc_acc)
            sc_acc = jnp.where(kio == k + 1, pick2v, sc_acc)
            lb_acc = jnp.where(kio == k, c1, lb_acc)
            lb_acc = jnp.where(kio == k + 1, pick2c, lb_acc)
            q_acc = jnp.where(kio == k, q1v, q_acc)
            q_acc = jnp.where(kio == k + 1, pick2q, q_acc)

        scores_ref[...] = sc_acc
        labels_ref[...] = lb_acc

        qio_k = jax.lax.broadcasted_iota(jnp.int32, (K, Q), 1)
        for b in range(B):
            qcol = jnp.reshape(q_acc[b], (K, 1))
            oh = (qio_k == qcol).astype(jnp.float32)
            boxesout_ref[b] = jax.lax.dot_general(
                oh, bx_ref[b], (((1,), (1,)), ((), ())),
                preferred_element_type=jnp.float32,
                precision=jax.lax.Precision.HIGHEST)


def kernel(pred_logits, pred_boxes, target_sizes, positive_map):
    grid = (NG + 1,)
    scores, labels, boxes = pl.pallas_call(
        _body,
        grid=grid,
        in_specs=[
            pl.BlockSpec((CQ, B, T), lambda g: (jnp.minimum(g, NG - 1), 0, 0)),
            pl.BlockSpec((B, 4, Q), lambda g: (0, 0, 0)),
            pl.BlockSpec(memory_space=pltpu.SMEM),
            pl.BlockSpec((C, T), lambda g: (0, 0)),
        ],
        out_specs=[
            pl.BlockSpec((B, K), lambda g: (0, 0)),
            pl.BlockSpec((B, K), lambda g: (0, 0)),
            pl.BlockSpec((B, K, 4), lambda g: (0, 0, 0)),
        ],
        out_shape=[
            jax.ShapeDtypeStruct((B, K), jnp.float32),
            jax.ShapeDtypeStruct((B, K), jnp.int32),
            jax.ShapeDtypeStruct((B, K, 4), jnp.float32),
        ],
        scratch_shapes=[
            pltpu.VMEM((QP * B, C), jnp.float32),
            pltpu.VMEM((B, QP), jnp.float32),
            pltpu.VMEM((B, 4, Q), jnp.float32),
            pltpu.VMEM((C, T), jnp.bfloat16),
        ],
    )(jnp.transpose(pred_logits, (1, 0, 2)),
      jnp.transpose(pred_boxes, (0, 2, 1)),
      target_sizes, positive_map)
    return (scores, labels, boxes)


# speedup vs baseline: 1.0020x; 1.0020x over previous
import jax
import jax.numpy as jnp
from jax.experimental import pallas as pl
from jax.experimental.pallas import tpu as pltpu

B = 8
Q = 900
T = 512
C = 400
K = 50
CQ = 128
NG = 8
QP = CQ * NG
BIG = 1 << 30


def _body(logits_ref, boxes_ref, ts_ref, wt_ref,
          scores_ref, labels_ref, boxesout_ref,
          p_ref, rm_ref, bx_ref):
    pid = pl.program_id(0)

    @pl.when(pid < NG)
    def _matmul_step():
        x = logits_ref[...].reshape(CQ * B, T)
        sig = jax.nn.sigmoid(x).astype(jnp.bfloat16)
        pm = wt_ref[...].astype(jnp.bfloat16)
        p = jax.lax.dot_general(
            sig, pm, (((1,), (1,)), ((), ())),
            preferred_element_type=jnp.float32)
        p_ref[pl.ds(pid * CQ * B, CQ * B), :] = p
        rmc = jnp.max(p.reshape(CQ, B, C), axis=2)
        rm_ref[:, pl.ds(pid * CQ, CQ)] = jnp.transpose(rmc)

    @pl.when(pid == NG)
    def _extract_step():
        pb = boxes_ref[...]
        cxy = pb[:, 0:2, :]
        wh2 = pb[:, 2:4, :] * 0.5
        xyxy = jnp.concatenate([cxy - wh2, cxy + wh2], axis=1)
        for b in range(B):
            h = ts_ref[b, 0]
            w = ts_ref[b, 1]
            scale = jnp.concatenate(
                [jnp.full((1, Q), w, jnp.float32),
                 jnp.full((1, Q), h, jnp.float32)] * 2, axis=0)
            bx_ref[b] = xyxy[b] * scale

        qio = jax.lax.broadcasted_iota(jnp.int32, (B, QP), 1)
        cio = jax.lax.broadcasted_iota(jnp.int32, (B, C), 1)
        kio = jax.lax.broadcasted_iota(jnp.int32, (B, K), 1)
        rm = rm_ref[...]
        rm = jnp.where(qio >= Q, -1.0, rm)
        sc_acc = jnp.zeros((B, K), jnp.float32)
        lb_acc = jnp.zeros((B, K), jnp.int32)
        q_acc = jnp.zeros((B, K), jnp.int32)

        for k2 in range(K // 2):
            k = 2 * k2
            m1 = jnp.max(rm, axis=1, keepdims=True)
            q1v = jnp.min(jnp.where(rm == m1, qio, BIG),
                          axis=1, keepdims=True)
            rme = jnp.where(qio == q1v, -2.0, rm)
            m2 = jnp.max(rme, axis=1, keepdims=True)
            q2v = jnp.min(jnp.where(rme == m2, qio, BIG),
                          axis=1, keepdims=True)
            rowsA = []
            rowsB = []
            rsA = []
            rsB = []
            for b in range(B):
                ra = q1v[b, 0] * B + b
                rb = q2v[b, 0] * B + b
                rsA.append(ra)
                rsB.append(rb)
                rowsA.append(p_ref[pl.ds(ra, 1), :])
                rowsB.append(p_ref[pl.ds(rb, 1), :])
            rowsA = jnp.concatenate(rowsA, axis=0)
            rowsB = jnp.concatenate(rowsB, axis=0)
            c1 = jnp.min(jnp.where(rowsA == m1, cio, BIG),
                         axis=1, keepdims=True)
            nrowA = jnp.where(cio == c1, -1.0, rowsA)
            nm1 = jnp.max(nrowA, axis=1, keepdims=True)
            c1p = jnp.min(jnp.where(nrowA == nm1, cio, BIG),
                          axis=1, keepdims=True)
            c2 = jnp.min(jnp.where(rowsB == m2, cio, BIG),
                         axis=1, keepdims=True)
            flag = (nm1 > m2) | ((nm1 == m2) & (q1v < q2v))
            pick2v = jnp.where(flag, nm1, m2)
            pick2c = jnp.where(flag, c1p, c2)
            pick2q = jnp.where(flag, q1v, q2v)
            rowAf = jnp.where((cio == c1) | (flag & (cio == c1p)),
                              -1.0, rowsA)
            rowBf = jnp.where((~flag) & (cio == c2), -1.0, rowsB)
            for b in range(B):
                p_ref[pl.ds(rsA[b], 1), :] = rowAf[b:b + 1, :]
                p_ref[pl.ds(rsB[b], 1), :] = rowBf[b:b + 1, :]
            rmA = jnp.max(rowAf, axis=1, keepdims=True)
            rmB = jnp.max(rowBf, axis=1, keepdims=True)
            rm = jnp.where(qio == q1v, rmA, rm)
            rm = jnp.where(qio == q2v, rmB, rm)
            sc_acc = jnp.where(kio == k, m1, sc_acc)
            sc_acc = jnp.where(kio == k + 1, pick2v, sc_acc)
            lb_acc = jnp.where(kio == k, c1, lb_acc)
            lb_acc = jnp.where(kio == k + 1, pick2c, lb_acc)
            q_acc = jnp.where(kio == k, q1v, q_acc)
            q_acc = jnp.where(kio == k + 1, pick2q, q_acc)

        scores_ref[...] = sc_acc
        labels_ref[...] = lb_acc

        qio_k = jax.lax.broadcasted_iota(jnp.int32, (K, Q), 1)
        for b in range(B):
            qcol = jnp.reshape(q_acc[b], (K, 1))
            oh = (qio_k == qcol).astype(jnp.float32)
            boxesout_ref[b] = jax.lax.dot_general(
                oh, bx_ref[b], (((1,), (1,)), ((), ())),
                preferred_element_type=jnp.float32,
                precision=jax.lax.Precision.HIGHEST)


def kernel(pred_logits, pred_boxes, target_sizes, positive_map):
    grid = (NG + 1,)
    scores, labels, boxes = pl.pallas_call(
        _body,
        grid=grid,
        in_specs=[
            pl.BlockSpec((CQ, B, T), lambda g: (jnp.minimum(g, NG - 1), 0, 0)),
            pl.BlockSpec((B, 4, Q), lambda g: (0, 0, 0)),
            pl.BlockSpec(memory_space=pltpu.SMEM),
            pl.BlockSpec((C, T), lambda g: (0, 0)),
        ],
        out_specs=[
            pl.BlockSpec((B, K), lambda g: (0, 0)),
            pl.BlockSpec((B, K), lambda g: (0, 0)),
            pl.BlockSpec((B, K, 4), lambda g: (0, 0, 0)),
        ],
        out_shape=[
            jax.ShapeDtypeStruct((B, K), jnp.float32),
            jax.ShapeDtypeStruct((B, K), jnp.int32),
            jax.ShapeDtypeStruct((B, K, 4), jnp.float32),
        ],
        scratch_shapes=[
            pltpu.VMEM((QP * B, C), jnp.float32),
            pltpu.VMEM((B, QP), jnp.float32),
            pltpu.VMEM((B, 4, Q), jnp.float32),
        ],
    )(jnp.transpose(pred_logits, (1, 0, 2)),
      jnp.transpose(pred_boxes, (0, 2, 1)),
      target_sizes, positive_map)
    return (scores, labels, boxes)
